# baseline (speedup 1.0000x reference)
import jax
import jax.numpy as jnp
from jax import lax
from jax.experimental import pallas as pl
from jax.experimental.pallas import tpu as pltpu
from jax.experimental.pallas import tpu_sc as plsc

V = 1000000
D = 64
B = 16384
L = 20
N = B * L

NC = 2
NS = 16
NW = NC * NS
CB = 128
N_PER_W = N // NW
CHUNKS = N_PER_W // CB
DW = D // 2


def _body(idx_hbm, w_hbm, s_hbm, m_hbm, out_hbm, idx_v, rows_v, s_v, m_v,
          stage_v, sem):
  wid = lax.axis_index("s") * NC + lax.axis_index("c")
  lanes = lax.iota(jnp.int32, 16)

  def chunk_body(ci, _):
    base = wid * N_PER_W + ci * CB
    pltpu.sync_copy(idx_hbm.at[pl.ds(base, CB)], idx_v)
    cw = pltpu.async_copy(w_hbm.at[idx_v], rows_v, sem)
    cs = pltpu.async_copy(s_hbm.at[idx_v], s_v, sem)
    cm = pltpu.async_copy(m_hbm.at[idx_v], m_v, sem)
    cw.wait()
    cs.wait()
    cm.wait()

    def row_body(r, _):
      w32 = rows_v[r]
      b0 = (w32 << 24) >> 24
      b1 = (w32 << 16) >> 24
      b2 = (w32 << 8) >> 24
      b3 = w32 >> 24
      rsplat = jnp.full((16,), 0, jnp.int32) + r
      sv = plsc.load_gather(s_v, [rsplat])
      mv = plsc.load_gather(m_v, [rsplat])
      o0 = sv * (b0.astype(jnp.float32) + mv)
      o1 = sv * (b1.astype(jnp.float32) + mv)
      o2 = sv * (b2.astype(jnp.float32) + mv)
      o3 = sv * (b3.astype(jnp.float32) + mv)
      p01 = plsc.bitcast(
          plsc.pack(o0, o1, format=plsc.PackFormat.INTERLEAVED), jnp.int32)
      p23 = plsc.bitcast(
          plsc.pack(o2, o3, format=plsc.PackFormat.INTERLEAVED), jnp.int32)
      wbase = r * DW + 2 * lanes
      plsc.store_scatter(stage_v, [wbase], p01)
      plsc.store_scatter(stage_v, [wbase + 1], p23)
      return 0

    lax.fori_loop(0, CB, row_body, 0)
    pltpu.sync_copy(stage_v, out_hbm.at[pl.ds(base * DW, CB * DW)])
    return 0

  lax.fori_loop(0, CHUNKS, chunk_body, 0)


@jax.jit
def kernel(idx, weight, scales, means):
  mesh = plsc.VectorSubcoreMesh(core_axis_name="c", subcore_axis_name="s")
  run = pl.kernel(
      _body,
      out_type=jax.ShapeDtypeStruct((N * DW,), jnp.int32),
      mesh=mesh,
      compiler_params=pltpu.CompilerParams(
          needs_layout_passes=False, use_tc_tiling_on_sc=False),
      scratch_types=[
          pltpu.VMEM((CB,), jnp.int32),
          pltpu.VMEM((CB, D // 4), jnp.int32),
          pltpu.VMEM((CB,), jnp.float32),
          pltpu.VMEM((CB,), jnp.float32),
          pltpu.VMEM((CB * DW,), jnp.int32),
          pltpu.SemaphoreType.DMA,
      ],
  )
  w32 = lax.bitcast_convert_type(weight.reshape(V, D // 4, 4), jnp.int32)
  out_i32 = run(idx.reshape(N), w32, scales.reshape(V), means.reshape(V))
  return lax.bitcast_convert_type(out_i32, jnp.bfloat16).reshape(B, L, D)


# speedup vs baseline: 1.2804x; 1.2804x over previous
import jax
import jax.numpy as jnp
from jax import lax
from jax.experimental import pallas as pl
from jax.experimental.pallas import tpu as pltpu
from jax.experimental.pallas import tpu_sc as plsc

V = 1000000
D = 64
B = 16384
L = 20
N = B * L

NC = 2
NS = 16
NW = NC * NS
CB = 128
N_PER_W = N // NW
CHUNKS = N_PER_W // CB

KB = 250
V_PER_W = V // NW
KCHUNKS = V_PER_W // KB


def _repack_body(w_hbm, out_hbm, in_v, out_v, sem):
  wid = lax.axis_index("s") * NC + lax.axis_index("c")

  def chunk(ci, _):
    r0 = wid * V_PER_W + ci * KB
    pltpu.sync_copy(w_hbm.at[pl.ds(r0, KB)], in_v)

    def row(r, _):
      out_v[r] = plsc.bitcast(in_v[r], jnp.int32)
      return 0

    lax.fori_loop(0, KB, row, 0)
    pltpu.sync_copy(out_v, out_hbm.at[pl.ds(r0, KB)])
    return 0

  lax.fori_loop(0, KCHUNKS, chunk, 0)


def _gather_body(idx_hbm, w_hbm, s_hbm, m_hbm, out_hbm, idx_v, rows_v, s_v,
                 m_v, stage_v, obuf_v, sem):
  wid = lax.axis_index("s") * NC + lax.axis_index("c")
  lanes = lax.iota(jnp.int32, 16)

  def chunk_body(ci, _):
    base = wid * N_PER_W + ci * CB
    pltpu.sync_copy(idx_hbm.at[pl.ds(base, CB)], idx_v)
    cw = pltpu.async_copy(w_hbm.at[idx_v], rows_v, sem)
    cs = pltpu.async_copy(s_hbm.at[idx_v], s_v, sem)
    cm = pltpu.async_copy(m_hbm.at[idx_v], m_v, sem)
    cw.wait()
    cs.wait()
    cm.wait()

    def row_body(r, _):
      w32 = rows_v[r]
      rsplat = jnp.full((16,), 0, jnp.int32) + r
      sv = plsc.load_gather(s_v, [rsplat])
      mv = plsc.load_gather(m_v, [rsplat])
      o0 = sv * (((w32 << 24) >> 24).astype(jnp.float32) + mv)
      o1 = sv * (((w32 << 16) >> 24).astype(jnp.float32) + mv)
      o2 = sv * (((w32 << 8) >> 24).astype(jnp.float32) + mv)
      o3 = sv * ((w32 >> 24).astype(jnp.float32) + mv)
      p01 = plsc.bitcast(
          plsc.pack(o0, o1, format=plsc.PackFormat.INTERLEAVED), jnp.int32)
      p23 = plsc.bitcast(
          plsc.pack(o2, o3, format=plsc.PackFormat.INTERLEAVED), jnp.int32)
      wbase = r * (D // 2) + 2 * lanes
      plsc.store_scatter(stage_v, [wbase], p01)
      plsc.store_scatter(stage_v, [wbase + 1], p23)
      return 0

    lax.fori_loop(0, CB, row_body, 0)

    def reorder_body(t, _):
      for j in range(2):
        r = 2 * t + j
        half0 = plsc.bitcast(stage_v[pl.ds(r * (D // 2), 16)], jnp.bfloat16)
        half1 = plsc.bitcast(stage_v[pl.ds(r * (D // 2) + 16, 16)],
                             jnp.bfloat16)
        obuf_v[2 * t + j, pl.ds(0, 32)] = half0
        obuf_v[2 * t + j, pl.ds(32, 32)] = half1
      return 0

    lax.fori_loop(0, CB // 2, reorder_body, 0)
    pltpu.sync_copy(
        obuf_v, out_hbm.at[pl.ds(pl.multiple_of(base, 128), CB)])
    return 0

  lax.fori_loop(0, CHUNKS, chunk_body, 0)


@jax.jit
def kernel(idx, weight, scales, means):
  mesh = plsc.VectorSubcoreMesh(core_axis_name="c", subcore_axis_name="s")
  params = pltpu.CompilerParams(
      needs_layout_passes=False, use_tc_tiling_on_sc=False)
  repack = pl.kernel(
      _repack_body,
      out_type=jax.ShapeDtypeStruct((V, D // 4), jnp.int32),
      mesh=mesh,
      compiler_params=params,
      scratch_types=[
          pltpu.VMEM((KB, D), jnp.int8),
          pltpu.VMEM((KB, D // 4), jnp.int32),
          pltpu.SemaphoreType.DMA,
      ],
  )
  gather = pl.kernel(
      _gather_body,
      out_type=jax.ShapeDtypeStruct((N, D), jnp.bfloat16),
      mesh=mesh,
      compiler_params=params,
      scratch_types=[
          pltpu.VMEM((CB,), jnp.int32),
          pltpu.VMEM((CB, D // 4), jnp.int32),
          pltpu.VMEM((CB,), jnp.float32),
          pltpu.VMEM((CB,), jnp.float32),
          pltpu.VMEM((CB * D // 2,), jnp.int32),
          pltpu.VMEM((CB, D), jnp.bfloat16),
          pltpu.SemaphoreType.DMA,
      ],
  )
  w32 = repack(weight)
  out = gather(idx.reshape(N), w32, scales.reshape(V), means.reshape(V))
  return out.reshape(B, L, D)
